# baseline (speedup 1.0000x reference)
import functools

import jax
import jax.numpy as jnp
from jax import lax
from jax.experimental import pallas as pl
from jax.experimental.pallas import tpu as pltpu
from jax.experimental.pallas import tpu_sc as plsc

N = 10000
E = 320000
D = 128

NC = 2
NS = 16
NW = NC * NS
EPW = E // NW
CHUNK = 128
NCHUNK = 80
EPWP = NCHUNK * CHUNK
NA = N + 8
RPT = 624
REM = N - RPT * NS
NPAIR = NCHUNK // 2
SHIFT = 14
MASK = (1 << SHIFT) - 1


def _sc_segment_sum(x, pk2, zrows):
    mesh = plsc.VectorSubcoreMesh(core_axis_name="c", subcore_axis_name="s")

    @functools.partial(
        pl.kernel,
        mesh=mesh,
        out_type=jax.ShapeDtypeStruct((NC, N, D), jnp.float32),
        scratch_types=[
            pltpu.VMEM((EPWP,), jnp.int32),
            pltpu.VMEM((8, CHUNK), jnp.int32),
            pltpu.VMEM((8, CHUNK), jnp.int32),
            pltpu.VMEM((CHUNK, D), jnp.float32),
            pltpu.VMEM((CHUNK, D), jnp.float32),
            pltpu.VMEM_SHARED((NA, D), jnp.float32),
            pltpu.SemaphoreType.DMA,
        ],
    )
    def k(x_hbm, pk_hbm, z_hbm, out_hbm, pk_v, set_a, set_b, rows_a, rows_b, acc, sem):
        cid = lax.axis_index("c")
        sid = lax.axis_index("s")
        wid = sid * NC + cid
        pltpu.sync_copy(z_hbm, acc.at[pl.ds(sid * RPT, RPT)])

        @pl.when(sid == NS - 1)
        def _zero_rem():
            pltpu.sync_copy(z_hbm.at[pl.ds(0, REM)], acc.at[pl.ds(RPT * NS, REM)])
        pltpu.sync_copy(pk_hbm.at[wid], pk_v)
        plsc.subcore_barrier()

        def unpack(j, dest):
            for t in range(CHUNK // 16):
                v = pk_v[pl.ds(j * CHUNK + t * 16, 16)]
                dest[0, pl.ds(t * 16, 16)] = lax.bitwise_and(v, MASK)
                dest[1, pl.ds(t * 16, 16)] = lax.shift_right_logical(v, SHIFT)

        unpack(0, set_a)
        pltpu.async_copy(x_hbm.at[set_a.at[0]], rows_a, sem)
        unpack(1, set_b)
        pltpu.async_copy(x_hbm.at[set_b.at[0]], rows_b, sem)

        def body(i, carry):
            j = 2 * i
            pltpu.make_async_copy(x_hbm.at[set_a.at[0]], rows_a, sem).wait()
            pltpu.sync_copy(rows_a, acc.at[set_a.at[1]], add=True)

            @pl.when(i < NPAIR - 1)
            def _next_a():
                unpack(j + 2, set_a)
                pltpu.async_copy(x_hbm.at[set_a.at[0]], rows_a, sem)

            pltpu.make_async_copy(x_hbm.at[set_b.at[0]], rows_b, sem).wait()
            pltpu.sync_copy(rows_b, acc.at[set_b.at[1]], add=True)

            @pl.when(i < NPAIR - 1)
            def _next_b():
                unpack(j + 3, set_b)
                pltpu.async_copy(x_hbm.at[set_b.at[0]], rows_b, sem)

            return carry

        lax.fori_loop(0, NPAIR, body, 0)
        plsc.subcore_barrier()
        pltpu.sync_copy(acc.at[pl.ds(sid * RPT, RPT)],
                        out_hbm.at[cid, pl.ds(sid * RPT, RPT)])

        @pl.when(sid == NS - 1)
        def _write_rem():
            pltpu.sync_copy(acc.at[pl.ds(RPT * NS, REM)],
                            out_hbm.at[cid, pl.ds(RPT * NS, REM)])

    return k(x, pk2, zrows)


_BLK = 1000


def _mlp1(x, p0, p1, scale, WaT, ba, WbT, bb):
    def body(x_r, p0_r, p1_r, s_r, wa_r, ba_r, wb_r, bb_r, o_r):
        h = x_r[...] * s_r[0, 0] + p0_r[...] + p1_r[...]
        h = jnp.maximum(
            jnp.dot(h, wa_r[...], preferred_element_type=jnp.float32,
                    precision=lax.Precision.HIGHEST) + ba_r[...], 0.0)
        h = jnp.maximum(
            jnp.dot(h, wb_r[...], preferred_element_type=jnp.float32,
                    precision=lax.Precision.HIGHEST) + bb_r[...], 0.0)
        o_r[...] = h

    row = pl.BlockSpec((_BLK, D), lambda i: (i, 0))
    full = pl.BlockSpec((D, D), lambda i: (0, 0))
    bias = pl.BlockSpec((1, D), lambda i: (0, 0))
    return pl.pallas_call(
        body,
        grid=(N // _BLK,),
        in_specs=[row, row, row, pl.BlockSpec((1, 1), lambda i: (0, 0)),
                  full, bias, full, bias],
        out_specs=row,
        out_shape=jax.ShapeDtypeStruct((N, D), jnp.float32),
    )(x, p0, p1, scale, WaT, ba, WbT, bb)


def _mlp2(h, q0, q1, WaT, ba, WlT, bl):
    def body(h_r, q0_r, q1_r, wa_r, ba_r, wl_r, bl_r, o_r):
        h2 = h_r[...] + q0_r[...] + q1_r[...]
        h2 = jnp.maximum(
            jnp.dot(h2, wa_r[...], preferred_element_type=jnp.float32,
                    precision=lax.Precision.HIGHEST) + ba_r[...], 0.0)
        o_r[...] = jnp.dot(h2, wl_r[...], preferred_element_type=jnp.float32,
                           precision=lax.Precision.HIGHEST) + bl_r[...]

    row = pl.BlockSpec((_BLK, D), lambda i: (i, 0))
    full = pl.BlockSpec((D, D), lambda i: (0, 0))
    bias = pl.BlockSpec((1, D), lambda i: (0, 0))
    return pl.pallas_call(
        body,
        grid=(N // _BLK,),
        in_specs=[row, row, row, full, bias, full, bias],
        out_specs=row,
        out_shape=jax.ShapeDtypeStruct((N, D), jnp.float32),
    )(h, q0, q1, WaT, ba, WlT, bl)


def kernel(x, edge_index, eps1, W1a, b1a, W1b, b1b, W2a, b2a, Wl, bl):
    pad = EPWP - EPW
    src2 = jnp.pad(edge_index[0].reshape(NW, EPW), ((0, 0), (0, pad)))
    dst2 = jnp.pad(edge_index[1].reshape(NW, EPW), ((0, 0), (0, pad)),
                   constant_values=N)
    pk2 = (dst2 << SHIFT) | src2
    zrows = jnp.zeros((RPT, D), jnp.float32)
    scale = jnp.reshape(1.0 + eps1, (1, 1))
    p = _sc_segment_sum(x, pk2, zrows)
    h = _mlp1(x, p[0], p[1], scale, W1a.T, b1a.reshape(1, D), W1b.T, b1b.reshape(1, D))
    q = _sc_segment_sum(h, pk2, zrows)
    return _mlp2(h, q[0], q[1], W2a.T, b2a.reshape(1, D), Wl.T, bl.reshape(1, D))


# speedup vs baseline: 2.5388x; 2.5388x over previous
import functools

import jax
import jax.numpy as jnp
from jax import lax
from jax.experimental import pallas as pl
from jax.experimental.pallas import tpu as pltpu
from jax.experimental.pallas import tpu_sc as plsc

N = 10000
E = 320000
D = 128

NC = 2
NS = 16
NW = NC * NS
EPW = E // NW
CHUNK = 80
NCHUNK = EPW // CHUNK
RPT = 624
REM = N - RPT * NS
NPAIR = NCHUNK // 2
SHIFT = 14
MASK = (1 << SHIFT) - 1


def _sc_segment_sum(x, pk2, zrows):
    mesh = plsc.VectorSubcoreMesh(core_axis_name="c", subcore_axis_name="s")

    @functools.partial(
        pl.kernel,
        mesh=mesh,
        out_type=jax.ShapeDtypeStruct((NC, N, D), jnp.float32),
        scratch_types=[
            pltpu.VMEM((EPW,), jnp.int32),
            pltpu.VMEM((8, CHUNK), jnp.int32),
            pltpu.VMEM((8, CHUNK), jnp.int32),
            pltpu.VMEM((CHUNK, D), jnp.float32),
            pltpu.VMEM((CHUNK, D), jnp.float32),
            pltpu.VMEM_SHARED((N, D), jnp.float32),
            pltpu.SemaphoreType.DMA,
        ],
    )
    def k(x_hbm, pk_hbm, z_hbm, out_hbm, pk_v, set_a, set_b, rows_a, rows_b, acc, sem):
        cid = lax.axis_index("c")
        sid = lax.axis_index("s")
        wid = sid * NC + cid
        pltpu.sync_copy(z_hbm, acc.at[pl.ds(sid * RPT, RPT)])

        @pl.when(sid == NS - 1)
        def _zero_rem():
            pltpu.sync_copy(z_hbm.at[pl.ds(0, REM)], acc.at[pl.ds(RPT * NS, REM)])
        pltpu.sync_copy(pk_hbm.at[wid], pk_v)
        plsc.subcore_barrier()

        def unpack(j, dest):
            for t in range(CHUNK // 16):
                v = pk_v[pl.ds(j * CHUNK + t * 16, 16)]
                dest[0, pl.ds(t * 16, 16)] = lax.bitwise_and(v, MASK)
                dest[1, pl.ds(t * 16, 16)] = lax.shift_right_logical(v, SHIFT)

        unpack(0, set_a)
        pltpu.async_copy(x_hbm.at[set_a.at[0]], rows_a, sem)
        unpack(1, set_b)
        pltpu.async_copy(x_hbm.at[set_b.at[0]], rows_b, sem)

        def body(i, carry):
            j = 2 * i
            pltpu.make_async_copy(x_hbm.at[set_a.at[0]], rows_a, sem).wait()
            pltpu.sync_copy(rows_a, acc.at[set_a.at[1]], add=True)
            unpack(j + 2, set_a)
            pltpu.async_copy(x_hbm.at[set_a.at[0]], rows_a, sem)

            pltpu.make_async_copy(x_hbm.at[set_b.at[0]], rows_b, sem).wait()
            pltpu.sync_copy(rows_b, acc.at[set_b.at[1]], add=True)

            @pl.when(i < NPAIR - 1)
            def _next_b():
                unpack(j + 3, set_b)
                pltpu.async_copy(x_hbm.at[set_b.at[0]], rows_b, sem)

            return carry

        lax.fori_loop(0, NPAIR, body, 0)
        pltpu.make_async_copy(x_hbm.at[set_a.at[0]], rows_a, sem).wait()
        pltpu.sync_copy(rows_a, acc.at[set_a.at[1]], add=True)
        plsc.subcore_barrier()
        pltpu.sync_copy(acc.at[pl.ds(sid * RPT, RPT)],
                        out_hbm.at[cid, pl.ds(sid * RPT, RPT)])

        @pl.when(sid == NS - 1)
        def _write_rem():
            pltpu.sync_copy(acc.at[pl.ds(RPT * NS, REM)],
                            out_hbm.at[cid, pl.ds(RPT * NS, REM)])

    return k(x, pk2, zrows)


_BLK = 1000


def _mlp1(x, p0, p1, scale, WaT, ba, WbT, bb):
    def body(x_r, p0_r, p1_r, s_r, wa_r, ba_r, wb_r, bb_r, o_r):
        h = x_r[...] * s_r[0, 0] + p0_r[...] + p1_r[...]
        h = jnp.maximum(
            jnp.dot(h, wa_r[...], preferred_element_type=jnp.float32,
                    precision=lax.Precision.HIGHEST) + ba_r[...], 0.0)
        h = jnp.maximum(
            jnp.dot(h, wb_r[...], preferred_element_type=jnp.float32,
                    precision=lax.Precision.HIGHEST) + bb_r[...], 0.0)
        o_r[...] = h

    row = pl.BlockSpec((_BLK, D), lambda i: (i, 0))
    full = pl.BlockSpec((D, D), lambda i: (0, 0))
    bias = pl.BlockSpec((1, D), lambda i: (0, 0))
    return pl.pallas_call(
        body,
        grid=(N // _BLK,),
        in_specs=[row, row, row, pl.BlockSpec((1, 1), lambda i: (0, 0)),
                  full, bias, full, bias],
        out_specs=row,
        out_shape=jax.ShapeDtypeStruct((N, D), jnp.float32),
    )(x, p0, p1, scale, WaT, ba, WbT, bb)


def _mlp2(h, q0, q1, WaT, ba, WlT, bl):
    def body(h_r, q0_r, q1_r, wa_r, ba_r, wl_r, bl_r, o_r):
        h2 = h_r[...] + q0_r[...] + q1_r[...]
        h2 = jnp.maximum(
            jnp.dot(h2, wa_r[...], preferred_element_type=jnp.float32,
                    precision=lax.Precision.HIGHEST) + ba_r[...], 0.0)
        o_r[...] = jnp.dot(h2, wl_r[...], preferred_element_type=jnp.float32,
                           precision=lax.Precision.HIGHEST) + bl_r[...]

    row = pl.BlockSpec((_BLK, D), lambda i: (i, 0))
    full = pl.BlockSpec((D, D), lambda i: (0, 0))
    bias = pl.BlockSpec((1, D), lambda i: (0, 0))
    return pl.pallas_call(
        body,
        grid=(N // _BLK,),
        in_specs=[row, row, row, full, bias, full, bias],
        out_specs=row,
        out_shape=jax.ShapeDtypeStruct((N, D), jnp.float32),
    )(h, q0, q1, WaT, ba, WlT, bl)


def kernel(x, edge_index, eps1, W1a, b1a, W1b, b1b, W2a, b2a, Wl, bl):
    src2 = edge_index[0].reshape(NW, EPW)
    dst2 = edge_index[1].reshape(NW, EPW)
    pk2 = (dst2 << SHIFT) | src2
    zrows = jnp.zeros((RPT, D), jnp.float32)
    scale = jnp.reshape(1.0 + eps1, (1, 1))
    p = _sc_segment_sum(x, pk2, zrows)
    h = _mlp1(x, p[0], p[1], scale, W1a.T, b1a.reshape(1, D), W1b.T, b1b.reshape(1, D))
    q = _sc_segment_sum(h, pk2, zrows)
    return _mlp2(h, q[0], q[1], W2a.T, b2a.reshape(1, D), Wl.T, bl.reshape(1, D))


# speedup vs baseline: 2.6308x; 1.0362x over previous
import functools

import jax
import jax.numpy as jnp
from jax import lax
from jax.experimental import pallas as pl
from jax.experimental.pallas import tpu as pltpu
from jax.experimental.pallas import tpu_sc as plsc

N = 10000
E = 320000
D = 128

NC = 2
NS = 16
NW = NC * NS
EPW = E // NW
CHUNK = 80
NCHUNK = EPW // CHUNK
RPT = 624
REM = N - RPT * NS
NPAIR = NCHUNK // 2


def _sc_segment_sum(x, src2, dst3, zrows):
    mesh = plsc.VectorSubcoreMesh(core_axis_name="c", subcore_axis_name="s")

    @functools.partial(
        pl.kernel,
        mesh=mesh,
        out_type=jax.ShapeDtypeStruct((NC, N, D), jnp.float32),
        scratch_types=[
            pltpu.VMEM((EPW,), jnp.int32),
            pltpu.VMEM((NCHUNK, CHUNK), jnp.int32),
            pltpu.VMEM((CHUNK, D), jnp.float32),
            pltpu.VMEM((CHUNK, D), jnp.float32),
            pltpu.VMEM_SHARED((N, D), jnp.float32),
            pltpu.SemaphoreType.DMA,
        ],
    )
    def k(x_hbm, src_hbm, dst_hbm, z_hbm, out_hbm, src_v, dst_v, rows_a, rows_b, acc, sem):
        cid = lax.axis_index("c")
        sid = lax.axis_index("s")
        wid = sid * NC + cid
        pltpu.sync_copy(z_hbm, acc.at[pl.ds(sid * RPT, RPT)])

        @pl.when(sid == NS - 1)
        def _zero_rem():
            pltpu.sync_copy(z_hbm.at[pl.ds(0, REM)], acc.at[pl.ds(RPT * NS, REM)])
        pltpu.sync_copy(src_hbm.at[wid], src_v)
        pltpu.sync_copy(dst_hbm.at[wid], dst_v)
        plsc.subcore_barrier()

        def gidx(j):
            return src_v.at[pl.ds(j * CHUNK, CHUNK)]

        pltpu.async_copy(x_hbm.at[gidx(0)], rows_a, sem)
        pltpu.async_copy(x_hbm.at[gidx(1)], rows_b, sem)

        def body(i, carry):
            j = 2 * i
            pltpu.make_async_copy(x_hbm.at[gidx(j)], rows_a, sem).wait()
            pltpu.sync_copy(rows_a, acc.at[dst_v.at[j]], add=True)
            pltpu.async_copy(x_hbm.at[gidx(j + 2)], rows_a, sem)

            pltpu.make_async_copy(x_hbm.at[gidx(j + 1)], rows_b, sem).wait()
            pltpu.sync_copy(rows_b, acc.at[dst_v.at[j + 1]], add=True)

            @pl.when(i < NPAIR - 1)
            def _next_b():
                pltpu.async_copy(x_hbm.at[gidx(j + 3)], rows_b, sem)

            return carry

        lax.fori_loop(0, NPAIR, body, 0)
        pltpu.make_async_copy(x_hbm.at[gidx(NCHUNK - 1)], rows_a, sem).wait()
        pltpu.sync_copy(rows_a, acc.at[dst_v.at[NCHUNK - 1]], add=True)
        plsc.subcore_barrier()
        pltpu.sync_copy(acc.at[pl.ds(sid * RPT, RPT)],
                        out_hbm.at[cid, pl.ds(sid * RPT, RPT)])

        @pl.when(sid == NS - 1)
        def _write_rem():
            pltpu.sync_copy(acc.at[pl.ds(RPT * NS, REM)],
                            out_hbm.at[cid, pl.ds(RPT * NS, REM)])

    return k(x, src2, dst3, zrows)


_BLK = 1000


def _mlp1(x, p, scale, WaT, ba, WbT, bb):
    def body(x_r, p0_r, p1_r, s_r, wa_r, ba_r, wb_r, bb_r, o_r):
        h = x_r[...] * s_r[0, 0] + p0_r[0] + p1_r[0]
        h = jnp.maximum(
            jnp.dot(h, wa_r[...], preferred_element_type=jnp.float32,
                    precision=lax.Precision.HIGHEST) + ba_r[...], 0.0)
        h = jnp.maximum(
            jnp.dot(h, wb_r[...], preferred_element_type=jnp.float32,
                    precision=lax.Precision.HIGHEST) + bb_r[...], 0.0)
        o_r[...] = h

    row = pl.BlockSpec((_BLK, D), lambda i: (i, 0))
    par0 = pl.BlockSpec((1, _BLK, D), lambda i: (0, i, 0))
    par1 = pl.BlockSpec((1, _BLK, D), lambda i: (1, i, 0))
    full = pl.BlockSpec((D, D), lambda i: (0, 0))
    bias = pl.BlockSpec((1, D), lambda i: (0, 0))
    return pl.pallas_call(
        body,
        grid=(N // _BLK,),
        in_specs=[row, par0, par1, pl.BlockSpec((1, 1), lambda i: (0, 0)),
                  full, bias, full, bias],
        out_specs=row,
        out_shape=jax.ShapeDtypeStruct((N, D), jnp.float32),
    )(x, p, p, scale, WaT, ba, WbT, bb)


def _mlp2(h, q, WaT, ba, WlT, bl):
    def body(h_r, q0_r, q1_r, wa_r, ba_r, wl_r, bl_r, o_r):
        h2 = h_r[...] + q0_r[0] + q1_r[0]
        h2 = jnp.maximum(
            jnp.dot(h2, wa_r[...], preferred_element_type=jnp.float32,
                    precision=lax.Precision.HIGHEST) + ba_r[...], 0.0)
        o_r[...] = jnp.dot(h2, wl_r[...], preferred_element_type=jnp.float32,
                           precision=lax.Precision.HIGHEST) + bl_r[...]

    row = pl.BlockSpec((_BLK, D), lambda i: (i, 0))
    par0 = pl.BlockSpec((1, _BLK, D), lambda i: (0, i, 0))
    par1 = pl.BlockSpec((1, _BLK, D), lambda i: (1, i, 0))
    full = pl.BlockSpec((D, D), lambda i: (0, 0))
    bias = pl.BlockSpec((1, D), lambda i: (0, 0))
    return pl.pallas_call(
        body,
        grid=(N // _BLK,),
        in_specs=[row, par0, par1, full, bias, full, bias],
        out_specs=row,
        out_shape=jax.ShapeDtypeStruct((N, D), jnp.float32),
    )(h, q, q, WaT, ba, WlT, bl)


def kernel(x, edge_index, eps1, W1a, b1a, W1b, b1b, W2a, b2a, Wl, bl):
    src2 = edge_index[0].reshape(NW, EPW)
    dst3 = edge_index[1].reshape(NW, NCHUNK, CHUNK)
    zrows = jnp.zeros((RPT, D), jnp.float32)
    scale = jnp.reshape(1.0 + eps1, (1, 1))
    p = _sc_segment_sum(x, src2, dst3, zrows)
    h = _mlp1(x, p, scale, W1a.T, b1a.reshape(1, D), W1b.T, b1b.reshape(1, D))
    q = _sc_segment_sum(h, src2, dst3, zrows)
    return _mlp2(h, q, W2a.T, b2a.reshape(1, D), Wl.T, bl.reshape(1, D))


# speedup vs baseline: 2.8838x; 1.0962x over previous
import functools

import jax
import jax.numpy as jnp
from jax import lax
from jax.experimental import pallas as pl
from jax.experimental.pallas import tpu as pltpu
from jax.experimental.pallas import tpu_sc as plsc

N = 10000
E = 320000
D = 128

NC = 2
NS = 16
NW = NC * NS
EPW = E // NW
CHUNK = 80
NCHUNK = EPW // CHUNK
RPT = 624
REM = N - RPT * NS
NPAIR = NCHUNK // 2


def _sc_segment_sum(x, src2, dst3, zrows):
    mesh = plsc.VectorSubcoreMesh(core_axis_name="c", subcore_axis_name="s")

    @functools.partial(
        pl.kernel,
        mesh=mesh,
        out_type=jax.ShapeDtypeStruct((NC, N, D), jnp.float32),
        scratch_types=[
            pltpu.VMEM((EPW,), jnp.int32),
            pltpu.VMEM((NCHUNK, CHUNK), jnp.int32),
            pltpu.VMEM((CHUNK, D), jnp.float32),
            pltpu.VMEM((CHUNK, D), jnp.float32),
            pltpu.VMEM_SHARED((N, D), jnp.float32),
            pltpu.SemaphoreType.DMA,
        ],
    )
    def k(x_hbm, src_hbm, dst_hbm, z_hbm, out_hbm, src_v, dst_v, rows_a, rows_b, acc, sem):
        cid = lax.axis_index("c")
        sid = lax.axis_index("s")
        wid = sid * NC + cid
        pltpu.sync_copy(z_hbm, acc.at[pl.ds(sid * RPT, RPT)])

        @pl.when(sid == NS - 1)
        def _zero_rem():
            pltpu.sync_copy(z_hbm.at[pl.ds(0, REM)], acc.at[pl.ds(RPT * NS, REM)])
        pltpu.sync_copy(src_hbm.at[wid], src_v)
        pltpu.sync_copy(dst_hbm.at[wid], dst_v)
        plsc.subcore_barrier()

        def gidx(j):
            return src_v.at[pl.ds(j * CHUNK, CHUNK)]

        pltpu.async_copy(x_hbm.at[gidx(0)], rows_a, sem)
        pltpu.async_copy(x_hbm.at[gidx(1)], rows_b, sem)

        def body(i, carry):
            j = 2 * i
            pltpu.make_async_copy(x_hbm.at[gidx(j)], rows_a, sem).wait()
            pltpu.sync_copy(rows_a, acc.at[dst_v.at[j]], add=True)
            pltpu.async_copy(x_hbm.at[gidx(j + 2)], rows_a, sem)

            pltpu.make_async_copy(x_hbm.at[gidx(j + 1)], rows_b, sem).wait()
            pltpu.sync_copy(rows_b, acc.at[dst_v.at[j + 1]], add=True)

            @pl.when(i < NPAIR - 1)
            def _next_b():
                pltpu.async_copy(x_hbm.at[gidx(j + 3)], rows_b, sem)

            return carry

        lax.fori_loop(0, NPAIR, body, 0)
        pltpu.make_async_copy(x_hbm.at[gidx(NCHUNK - 1)], rows_a, sem).wait()
        pltpu.sync_copy(rows_a, acc.at[dst_v.at[NCHUNK - 1]], add=True)
        plsc.subcore_barrier()
        pltpu.sync_copy(acc.at[pl.ds(sid * RPT, RPT)],
                        out_hbm.at[cid, pl.ds(sid * RPT, RPT)])

        @pl.when(sid == NS - 1)
        def _write_rem():
            pltpu.sync_copy(acc.at[pl.ds(RPT * NS, REM)],
                            out_hbm.at[cid, pl.ds(RPT * NS, REM)])

    return k(x, src2, dst3, zrows)


_BLK = 2000


def _mlp1(x, p, scale, WaT, ba, WbT, bb):
    def body(x_r, p0_r, p1_r, s_r, wa_r, ba_r, wb_r, bb_r, o_r):
        h = x_r[...] * s_r[0, 0] + p0_r[0] + p1_r[0]
        h = jnp.maximum(
            jnp.dot(h, wa_r[...], preferred_element_type=jnp.float32,
                    precision=lax.Precision.HIGHEST) + ba_r[...], 0.0)
        h = jnp.maximum(
            jnp.dot(h, wb_r[...], preferred_element_type=jnp.float32,
                    precision=lax.Precision.HIGHEST) + bb_r[...], 0.0)
        o_r[...] = h

    row = pl.BlockSpec((_BLK, D), lambda i: (i, 0))
    par0 = pl.BlockSpec((1, _BLK, D), lambda i: (0, i, 0))
    par1 = pl.BlockSpec((1, _BLK, D), lambda i: (1, i, 0))
    full = pl.BlockSpec((D, D), lambda i: (0, 0))
    bias = pl.BlockSpec((1, D), lambda i: (0, 0))
    return pl.pallas_call(
        body,
        grid=(N // _BLK,),
        in_specs=[row, par0, par1, pl.BlockSpec((1, 1), lambda i: (0, 0)),
                  full, bias, full, bias],
        out_specs=row,
        out_shape=jax.ShapeDtypeStruct((N, D), jnp.float32),
    )(x, p, p, scale, WaT, ba, WbT, bb)


def _mlp2(h, q, WaT, ba, WlT, bl):
    def body(h_r, q0_r, q1_r, wa_r, ba_r, wl_r, bl_r, o_r):
        h2 = h_r[...] + q0_r[0] + q1_r[0]
        h2 = jnp.maximum(
            jnp.dot(h2, wa_r[...], preferred_element_type=jnp.float32,
                    precision=lax.Precision.HIGHEST) + ba_r[...], 0.0)
        o_r[...] = jnp.dot(h2, wl_r[...], preferred_element_type=jnp.float32,
                           precision=lax.Precision.HIGHEST) + bl_r[...]

    row = pl.BlockSpec((_BLK, D), lambda i: (i, 0))
    par0 = pl.BlockSpec((1, _BLK, D), lambda i: (0, i, 0))
    par1 = pl.BlockSpec((1, _BLK, D), lambda i: (1, i, 0))
    full = pl.BlockSpec((D, D), lambda i: (0, 0))
    bias = pl.BlockSpec((1, D), lambda i: (0, 0))
    return pl.pallas_call(
        body,
        grid=(N // _BLK,),
        in_specs=[row, par0, par1, full, bias, full, bias],
        out_specs=row,
        out_shape=jax.ShapeDtypeStruct((N, D), jnp.float32),
    )(h, q, q, WaT, ba, WlT, bl)


def kernel(x, edge_index, eps1, W1a, b1a, W1b, b1b, W2a, b2a, Wl, bl):
    src2 = edge_index[0].reshape(NW, EPW)
    dst3 = edge_index[1].reshape(NW, NCHUNK, CHUNK)
    zrows = jnp.zeros((RPT, D), jnp.float32)
    scale = jnp.reshape(1.0 + eps1, (1, 1))
    p = _sc_segment_sum(x, src2, dst3, zrows)
    h = _mlp1(x, p, scale, W1a.T, b1a.reshape(1, D), W1b.T, b1b.reshape(1, D))
    q = _sc_segment_sum(h, src2, dst3, zrows)
    return _mlp2(h, q, W2a.T, b2a.reshape(1, D), Wl.T, bl.reshape(1, D))


# speedup vs baseline: 3.0741x; 1.0660x over previous
import functools

import jax
import jax.numpy as jnp
from jax import lax
from jax.experimental import pallas as pl
from jax.experimental.pallas import tpu as pltpu
from jax.experimental.pallas import tpu_sc as plsc

N = 10000
E = 320000
D = 128

NC = 2
NS = 16
NW = NC * NS
EPW = E // NW
CHUNK = 80
NCHUNK = EPW // CHUNK
RPT = 624
REM = N - RPT * NS
NPAIR = NCHUNK // 2


def _sc_segment_sum(x, src2, dst3, zrows):
    mesh = plsc.VectorSubcoreMesh(core_axis_name="c", subcore_axis_name="s")

    @functools.partial(
        pl.kernel,
        mesh=mesh,
        out_type=jax.ShapeDtypeStruct((NC, N, D), jnp.float32),
        scratch_types=[
            pltpu.VMEM((EPW,), jnp.int32),
            pltpu.VMEM((NCHUNK, CHUNK), jnp.int32),
            pltpu.VMEM((CHUNK, D), jnp.float32),
            pltpu.VMEM((CHUNK, D), jnp.float32),
            pltpu.VMEM_SHARED((N, D), jnp.float32),
            pltpu.SemaphoreType.DMA,
        ],
    )
    def k(x_hbm, src_hbm, dst_hbm, z_hbm, out_hbm, src_v, dst_v, rows_a, rows_b, acc, sem):
        cid = lax.axis_index("c")
        sid = lax.axis_index("s")
        wid = sid * NC + cid
        pltpu.sync_copy(z_hbm, acc.at[pl.ds(sid * RPT, RPT)])

        @pl.when(sid == NS - 1)
        def _zero_rem():
            pltpu.sync_copy(z_hbm.at[pl.ds(0, REM)], acc.at[pl.ds(RPT * NS, REM)])
        pltpu.sync_copy(src_hbm.at[wid], src_v)
        pltpu.sync_copy(dst_hbm.at[wid], dst_v)
        plsc.subcore_barrier()

        def gidx(j):
            return src_v.at[pl.ds(j * CHUNK, CHUNK)]

        pltpu.async_copy(x_hbm.at[gidx(0)], rows_a, sem)
        pltpu.async_copy(x_hbm.at[gidx(1)], rows_b, sem)

        def body(i, carry):
            j = 2 * i
            pltpu.make_async_copy(x_hbm.at[gidx(j)], rows_a, sem).wait()
            pltpu.sync_copy(rows_a, acc.at[dst_v.at[j]], add=True)
            pltpu.async_copy(x_hbm.at[gidx(j + 2)], rows_a, sem)

            pltpu.make_async_copy(x_hbm.at[gidx(j + 1)], rows_b, sem).wait()
            pltpu.sync_copy(rows_b, acc.at[dst_v.at[j + 1]], add=True)

            @pl.when(i < NPAIR - 1)
            def _next_b():
                pltpu.async_copy(x_hbm.at[gidx(j + 3)], rows_b, sem)

            return carry

        lax.fori_loop(0, NPAIR, body, 0)
        pltpu.make_async_copy(x_hbm.at[gidx(NCHUNK - 1)], rows_a, sem).wait()
        pltpu.sync_copy(rows_a, acc.at[dst_v.at[NCHUNK - 1]], add=True)
        plsc.subcore_barrier()
        pltpu.sync_copy(acc.at[pl.ds(sid * RPT, RPT)],
                        out_hbm.at[cid, pl.ds(sid * RPT, RPT)])

        @pl.when(sid == NS - 1)
        def _write_rem():
            pltpu.sync_copy(acc.at[pl.ds(RPT * NS, REM)],
                            out_hbm.at[cid, pl.ds(RPT * NS, REM)])

    return k(x, src2, dst3, zrows)


_BLK = 2000


def _mlp1(x, p, scale, WaT, ba, WbT, bb):
    def body(x_r, p0_r, p1_r, s_r, wa_r, ba_r, wb_r, bb_r, o_r):
        h = x_r[...] * s_r[0, 0] + p0_r[0] + p1_r[0]
        h = jnp.maximum(
            jnp.dot(h, wa_r[...], preferred_element_type=jnp.float32) + ba_r[...], 0.0)
        h = jnp.maximum(
            jnp.dot(h, wb_r[...], preferred_element_type=jnp.float32) + bb_r[...], 0.0)
        o_r[...] = h

    row = pl.BlockSpec((_BLK, D), lambda i: (i, 0))
    par0 = pl.BlockSpec((1, _BLK, D), lambda i: (0, i, 0))
    par1 = pl.BlockSpec((1, _BLK, D), lambda i: (1, i, 0))
    full = pl.BlockSpec((D, D), lambda i: (0, 0))
    bias = pl.BlockSpec((1, D), lambda i: (0, 0))
    return pl.pallas_call(
        body,
        grid=(N // _BLK,),
        in_specs=[row, par0, par1, pl.BlockSpec((1, 1), lambda i: (0, 0)),
                  full, bias, full, bias],
        out_specs=row,
        out_shape=jax.ShapeDtypeStruct((N, D), jnp.float32),
    )(x, p, p, scale, WaT, ba, WbT, bb)


def _mlp2(h, q, WaT, ba, WlT, bl):
    def body(h_r, q0_r, q1_r, wa_r, ba_r, wl_r, bl_r, o_r):
        h2 = h_r[...] + q0_r[0] + q1_r[0]
        h2 = jnp.maximum(
            jnp.dot(h2, wa_r[...], preferred_element_type=jnp.float32) + ba_r[...], 0.0)
        o_r[...] = jnp.dot(h2, wl_r[...], preferred_element_type=jnp.float32) + bl_r[...]

    row = pl.BlockSpec((_BLK, D), lambda i: (i, 0))
    par0 = pl.BlockSpec((1, _BLK, D), lambda i: (0, i, 0))
    par1 = pl.BlockSpec((1, _BLK, D), lambda i: (1, i, 0))
    full = pl.BlockSpec((D, D), lambda i: (0, 0))
    bias = pl.BlockSpec((1, D), lambda i: (0, 0))
    return pl.pallas_call(
        body,
        grid=(N // _BLK,),
        in_specs=[row, par0, par1, full, bias, full, bias],
        out_specs=row,
        out_shape=jax.ShapeDtypeStruct((N, D), jnp.float32),
    )(h, q, q, WaT, ba, WlT, bl)


def kernel(x, edge_index, eps1, W1a, b1a, W1b, b1b, W2a, b2a, Wl, bl):
    src2 = edge_index[0].reshape(NW, EPW)
    dst3 = edge_index[1].reshape(NW, NCHUNK, CHUNK)
    zrows = jnp.zeros((RPT, D), jnp.float32)
    scale = jnp.reshape(1.0 + eps1, (1, 1))
    p = _sc_segment_sum(x, src2, dst3, zrows)
    h = _mlp1(x, p, scale, W1a.T, b1a.reshape(1, D), W1b.T, b1b.reshape(1, D))
    q = _sc_segment_sum(h, src2, dst3, zrows)
    return _mlp2(h, q, W2a.T, b2a.reshape(1, D), Wl.T, bl.reshape(1, D))


# speedup vs baseline: 3.5800x; 1.1646x over previous
import functools

import jax
import jax.numpy as jnp
from jax import lax
from jax.experimental import pallas as pl
from jax.experimental.pallas import tpu as pltpu
from jax.experimental.pallas import tpu_sc as plsc

N = 10000
E = 320000
D = 128

NC = 2
NS = 16
NW = NC * NS
EPW = E // NW
CHUNK = 80
NCHUNK = EPW // CHUNK
RPT = 624
REM = N - RPT * NS
NTRI = (NCHUNK - 2) // 3


def _sc_segment_sum(x, src2, dst1, zrows):
    mesh = plsc.VectorSubcoreMesh(core_axis_name="c", subcore_axis_name="s")

    @functools.partial(
        pl.kernel,
        mesh=mesh,
        out_type=jax.ShapeDtypeStruct((NC, N, D), jnp.float32),
        scratch_types=[
            pltpu.VMEM((EPW,), jnp.int32),
            pltpu.VMEM((24, CHUNK), jnp.int32),
            pltpu.VMEM((CHUNK, D), jnp.float32),
            pltpu.VMEM((CHUNK, D), jnp.float32),
            pltpu.VMEM((CHUNK, D), jnp.float32),
            pltpu.VMEM_SHARED((N, D), jnp.float32),
            pltpu.SemaphoreType.DMA,
            pltpu.SemaphoreType.DMA,
            pltpu.SemaphoreType.DMA,
        ],
    )
    def k(x_hbm, src_hbm, dst_hbm, z_hbm, out_hbm,
          src_v, sets_v, rows0, rows1, rows2, acc, sem_g, sem_s, sem_i):
        cid = lax.axis_index("c")
        sid = lax.axis_index("s")
        wid = sid * NC + cid
        rows = (rows0, rows1, rows2)
        base = wid * EPW

        def gidx(m):
            return src_v.at[pl.ds(m * CHUNK, CHUNK)]

        def didx(slot):
            return sets_v.at[8 * slot]

        def e_idx(m, slot):
            pltpu.async_copy(dst_hbm.at[pl.ds(base + m * CHUNK, CHUNK)],
                             didx(slot), sem_i)

        def w_idx(m, slot):
            pltpu.make_async_copy(dst_hbm.at[pl.ds(base + m * CHUNK, CHUNK)],
                                  didx(slot), sem_i).wait()

        def e_gat(m, slot):
            pltpu.async_copy(x_hbm.at[gidx(m)], rows[slot], sem_g)

        def w_gat(m, slot):
            pltpu.make_async_copy(x_hbm.at[gidx(m)], rows[slot], sem_g).wait()

        def e_sca(m, slot):
            pltpu.async_copy(rows[slot], acc.at[didx(slot)], sem_s, add=True)

        def w_sca(m, slot):
            pltpu.make_async_copy(rows[slot], acc.at[didx(slot)], sem_s).wait()

        pltpu.sync_copy(z_hbm, acc.at[pl.ds(sid * RPT, RPT)])

        @pl.when(sid == NS - 1)
        def _zero_rem():
            pltpu.sync_copy(z_hbm.at[pl.ds(0, REM)], acc.at[pl.ds(RPT * NS, REM)])
        pltpu.sync_copy(src_hbm.at[wid], src_v)
        e_idx(0, 0)
        e_idx(1, 1)
        plsc.subcore_barrier()
        e_gat(0, 0)
        e_gat(1, 1)

        def stage(m, slot, first):
            w_gat(m, slot)
            w_idx(m, slot)
            e_sca(m, slot)
            prev_slot = (slot + 2) % 3
            if not first:
                w_sca(m - 1, prev_slot)

            @pl.when(m + 2 < NCHUNK)
            def _prefetch():
                e_idx(m + 2, prev_slot)
                e_gat(m + 2, prev_slot)

        stage(0, 0, True)

        def body(i, carry):
            m = 3 * i
            stage(m + 1, 1, False)
            stage(m + 2, 2, False)
            stage(m + 3, 0, False)
            return carry

        lax.fori_loop(0, NTRI, body, 0)
        m0 = 3 * NTRI + 1
        w_gat(m0, 1)
        w_idx(m0, 1)
        e_sca(m0, 1)
        w_sca(m0 - 1, 0)
        w_sca(m0, 1)
        plsc.subcore_barrier()
        pltpu.sync_copy(acc.at[pl.ds(sid * RPT, RPT)],
                        out_hbm.at[cid, pl.ds(sid * RPT, RPT)])

        @pl.when(sid == NS - 1)
        def _write_rem():
            pltpu.sync_copy(acc.at[pl.ds(RPT * NS, REM)],
                            out_hbm.at[cid, pl.ds(RPT * NS, REM)])

    return k(x, src2, dst1, zrows)


_BLK = 2000


def _mlp1(x, p, scale, WaT, ba, WbT, bb):
    def body(x_r, p0_r, p1_r, s_r, wa_r, ba_r, wb_r, bb_r, o_r):
        h = x_r[...] * s_r[0, 0] + p0_r[0] + p1_r[0]
        h = jnp.maximum(
            jnp.dot(h, wa_r[...], preferred_element_type=jnp.float32) + ba_r[...], 0.0)
        h = jnp.maximum(
            jnp.dot(h, wb_r[...], preferred_element_type=jnp.float32) + bb_r[...], 0.0)
        o_r[...] = h

    row = pl.BlockSpec((_BLK, D), lambda i: (i, 0))
    par0 = pl.BlockSpec((1, _BLK, D), lambda i: (0, i, 0))
    par1 = pl.BlockSpec((1, _BLK, D), lambda i: (1, i, 0))
    full = pl.BlockSpec((D, D), lambda i: (0, 0))
    bias = pl.BlockSpec((1, D), lambda i: (0, 0))
    return pl.pallas_call(
        body,
        grid=(N // _BLK,),
        in_specs=[row, par0, par1, pl.BlockSpec((1, 1), lambda i: (0, 0)),
                  full, bias, full, bias],
        out_specs=row,
        out_shape=jax.ShapeDtypeStruct((N, D), jnp.float32),
    )(x, p, p, scale, WaT, ba, WbT, bb)


def _mlp2(h, q, WaT, ba, WlT, bl):
    def body(h_r, q0_r, q1_r, wa_r, ba_r, wl_r, bl_r, o_r):
        h2 = h_r[...] + q0_r[0] + q1_r[0]
        h2 = jnp.maximum(
            jnp.dot(h2, wa_r[...], preferred_element_type=jnp.float32) + ba_r[...], 0.0)
        o_r[...] = jnp.dot(h2, wl_r[...], preferred_element_type=jnp.float32) + bl_r[...]

    row = pl.BlockSpec((_BLK, D), lambda i: (i, 0))
    par0 = pl.BlockSpec((1, _BLK, D), lambda i: (0, i, 0))
    par1 = pl.BlockSpec((1, _BLK, D), lambda i: (1, i, 0))
    full = pl.BlockSpec((D, D), lambda i: (0, 0))
    bias = pl.BlockSpec((1, D), lambda i: (0, 0))
    return pl.pallas_call(
        body,
        grid=(N // _BLK,),
        in_specs=[row, par0, par1, full, bias, full, bias],
        out_specs=row,
        out_shape=jax.ShapeDtypeStruct((N, D), jnp.float32),
    )(h, q, q, WaT, ba, WlT, bl)


def kernel(x, edge_index, eps1, W1a, b1a, W1b, b1b, W2a, b2a, Wl, bl):
    src2 = edge_index[0].reshape(NW, EPW)
    dst1 = edge_index[1]
    zrows = jnp.zeros((RPT, D), jnp.float32)
    scale = jnp.reshape(1.0 + eps1, (1, 1))
    p = _sc_segment_sum(x, src2, dst1, zrows)
    h = _mlp1(x, p, scale, W1a.T, b1a.reshape(1, D), W1b.T, b1b.reshape(1, D))
    q = _sc_segment_sum(h, src2, dst1, zrows)
    return _mlp2(h, q, W2a.T, b2a.reshape(1, D), Wl.T, bl.reshape(1, D))


# speedup vs baseline: 3.6034x; 1.0065x over previous
import functools

import jax
import jax.numpy as jnp
from jax import lax
from jax.experimental import pallas as pl
from jax.experimental.pallas import tpu as pltpu
from jax.experimental.pallas import tpu_sc as plsc

N = 10000
E = 320000
D = 128

NC = 2
NS = 16
NW = NC * NS
EPW = E // NW
CHUNK = 80
NCHUNK = EPW // CHUNK
RPT = 624
REM = N - RPT * NS
NTRI = (NCHUNK - 2) // 3


def _sc_segment_sum(x, src2, dst1, zrows):
    mesh = plsc.VectorSubcoreMesh(core_axis_name="c", subcore_axis_name="s")

    @functools.partial(
        pl.kernel,
        mesh=mesh,
        out_type=jax.ShapeDtypeStruct((NC, N, D), jnp.float32),
        scratch_types=[
            pltpu.VMEM((EPW,), jnp.int32),
            pltpu.VMEM((24, CHUNK), jnp.int32),
            pltpu.VMEM((CHUNK, D), jnp.float32),
            pltpu.VMEM((CHUNK, D), jnp.float32),
            pltpu.VMEM((CHUNK, D), jnp.float32),
            pltpu.VMEM_SHARED((N, D), jnp.float32),
            pltpu.SemaphoreType.DMA,
            pltpu.SemaphoreType.DMA,
            pltpu.SemaphoreType.DMA,
        ],
    )
    def k(x_hbm, src_hbm, dst_hbm, z_hbm, out_hbm,
          src_v, sets_v, rows0, rows1, rows2, acc, sem_g, sem_s, sem_i):
        cid = lax.axis_index("c")
        sid = lax.axis_index("s")
        wid = sid * NC + cid
        rows = (rows0, rows1, rows2)
        base = wid * EPW

        def gidx(m):
            return src_v.at[pl.ds(m * CHUNK, CHUNK)]

        def didx(slot):
            return sets_v.at[8 * slot]

        def e_idx(m, slot):
            pltpu.async_copy(dst_hbm.at[pl.ds(base + m * CHUNK, CHUNK)],
                             didx(slot), sem_i)

        def w_idx(m, slot):
            pltpu.make_async_copy(dst_hbm.at[pl.ds(base + m * CHUNK, CHUNK)],
                                  didx(slot), sem_i).wait()

        def e_gat(m, slot):
            pltpu.async_copy(x_hbm.at[gidx(m)], rows[slot], sem_g)

        def w_gat(m, slot):
            pltpu.make_async_copy(x_hbm.at[gidx(m)], rows[slot], sem_g).wait()

        def e_sca(m, slot):
            pltpu.async_copy(rows[slot], acc.at[didx(slot)], sem_s, add=True)

        def w_sca(m, slot):
            pltpu.make_async_copy(rows[slot], acc.at[didx(slot)], sem_s).wait()

        pltpu.sync_copy(src_hbm.at[wid], src_v)
        e_idx(0, 0)
        e_idx(1, 1)
        e_gat(0, 0)
        e_gat(1, 1)
        pltpu.sync_copy(z_hbm, acc.at[pl.ds(sid * RPT, RPT)])

        @pl.when(sid == NS - 1)
        def _zero_rem():
            pltpu.sync_copy(z_hbm.at[pl.ds(0, REM)], acc.at[pl.ds(RPT * NS, REM)])
        plsc.subcore_barrier()

        def stage(m, slot, first):
            w_gat(m, slot)
            w_idx(m, slot)
            e_sca(m, slot)
            prev_slot = (slot + 2) % 3
            if not first:
                w_sca(m - 1, prev_slot)

            @pl.when(m + 2 < NCHUNK)
            def _prefetch():
                e_idx(m + 2, prev_slot)
                e_gat(m + 2, prev_slot)

        stage(0, 0, True)

        def body(i, carry):
            m = 3 * i
            stage(m + 1, 1, False)
            stage(m + 2, 2, False)
            stage(m + 3, 0, False)
            return carry

        lax.fori_loop(0, NTRI, body, 0)
        m0 = 3 * NTRI + 1
        w_gat(m0, 1)
        w_idx(m0, 1)
        e_sca(m0, 1)
        w_sca(m0 - 1, 0)
        w_sca(m0, 1)
        plsc.subcore_barrier()
        pltpu.sync_copy(acc.at[pl.ds(sid * RPT, RPT)],
                        out_hbm.at[cid, pl.ds(sid * RPT, RPT)])

        @pl.when(sid == NS - 1)
        def _write_rem():
            pltpu.sync_copy(acc.at[pl.ds(RPT * NS, REM)],
                            out_hbm.at[cid, pl.ds(RPT * NS, REM)])

    return k(x, src2, dst1, zrows)


_BLK = 2000


def _mlp1(x, p, scale, WaT, ba, WbT, bb):
    def body(x_r, p0_r, p1_r, s_r, wa_r, ba_r, wb_r, bb_r, o_r):
        h = x_r[...] * s_r[0, 0] + p0_r[0] + p1_r[0]
        h = jnp.maximum(
            jnp.dot(h, wa_r[...], preferred_element_type=jnp.float32) + ba_r[...], 0.0)
        h = jnp.maximum(
            jnp.dot(h, wb_r[...], preferred_element_type=jnp.float32) + bb_r[...], 0.0)
        o_r[...] = h

    row = pl.BlockSpec((_BLK, D), lambda i: (i, 0))
    par0 = pl.BlockSpec((1, _BLK, D), lambda i: (0, i, 0))
    par1 = pl.BlockSpec((1, _BLK, D), lambda i: (1, i, 0))
    full = pl.BlockSpec((D, D), lambda i: (0, 0))
    bias = pl.BlockSpec((1, D), lambda i: (0, 0))
    return pl.pallas_call(
        body,
        grid=(N // _BLK,),
        in_specs=[row, par0, par1, pl.BlockSpec((1, 1), lambda i: (0, 0)),
                  full, bias, full, bias],
        out_specs=row,
        out_shape=jax.ShapeDtypeStruct((N, D), jnp.float32),
    )(x, p, p, scale, WaT, ba, WbT, bb)


def _mlp2(h, q, WaT, ba, WlT, bl):
    def body(h_r, q0_r, q1_r, wa_r, ba_r, wl_r, bl_r, o_r):
        h2 = h_r[...] + q0_r[0] + q1_r[0]
        h2 = jnp.maximum(
            jnp.dot(h2, wa_r[...], preferred_element_type=jnp.float32) + ba_r[...], 0.0)
        o_r[...] = jnp.dot(h2, wl_r[...], preferred_element_type=jnp.float32) + bl_r[...]

    row = pl.BlockSpec((_BLK, D), lambda i: (i, 0))
    par0 = pl.BlockSpec((1, _BLK, D), lambda i: (0, i, 0))
    par1 = pl.BlockSpec((1, _BLK, D), lambda i: (1, i, 0))
    full = pl.BlockSpec((D, D), lambda i: (0, 0))
    bias = pl.BlockSpec((1, D), lambda i: (0, 0))
    return pl.pallas_call(
        body,
        grid=(N // _BLK,),
        in_specs=[row, par0, par1, full, bias, full, bias],
        out_specs=row,
        out_shape=jax.ShapeDtypeStruct((N, D), jnp.float32),
    )(h, q, q, WaT, ba, WlT, bl)


def kernel(x, edge_index, eps1, W1a, b1a, W1b, b1b, W2a, b2a, Wl, bl):
    src2 = edge_index[0].reshape(NW, EPW)
    dst1 = edge_index[1]
    zrows = jnp.zeros((RPT, D), jnp.float32)
    scale = jnp.reshape(1.0 + eps1, (1, 1))
    p = _sc_segment_sum(x, src2, dst1, zrows)
    h = _mlp1(x, p, scale, W1a.T, b1a.reshape(1, D), W1b.T, b1b.reshape(1, D))
    q = _sc_segment_sum(h, src2, dst1, zrows)
    return _mlp2(h, q, W2a.T, b2a.reshape(1, D), Wl.T, bl.reshape(1, D))
